# baseline (speedup 1.0000x reference)
import functools

import jax
import jax.numpy as jnp
from jax import lax
from jax.experimental import pallas as pl
from jax.experimental.pallas import tpu as pltpu
from jax.experimental.pallas import tpu_sc as plsc

_BN_EPS = 1e-3
_SQRT_HALF = 0.7071067811865476

_NC, _NS, _L = 2, 16, 16
_NW = _NC * _NS
_B = 128
_RT = 80
_CR = 5
_H = 32
_HP = 40


def _gelu(x):
    return x * 0.5 * (1.0 + lax.erf(x * _SQRT_HALF))


def _fold_layer(p):
    scale = p["gamma"] / jnp.sqrt(p["var"] + _BN_EPS)
    shift = p["beta"] - p["mean"] * scale
    w = scale[:, None] * p["W"]
    b = shift @ p["W"] + p["b"]
    return w, b


def _tc1_body(f_ref, w1_ref, b1_ref, w2_ref, b2_ref, ua_ref, u1_ref,
              msgs_ref, part_ref):
    x = f_ref[...]
    n = x.shape[0]
    h1 = _gelu(jnp.dot(x, w1_ref[...], preferred_element_type=jnp.float32)
               + b1_ref[...])
    m = _gelu(jnp.dot(h1, w2_ref[...], preferred_element_type=jnp.float32)
              + b2_ref[...])
    col = lax.broadcasted_iota(jnp.int32, (n, _HP - _H), 1)
    tail = jnp.where(col == 0, 1.0, 0.0).astype(jnp.float32)
    msgs_ref[...] = jnp.concatenate([m, tail], axis=1)
    part_ref[...] = (jnp.dot(x, ua_ref[...], preferred_element_type=jnp.float32)
                     + u1_ref[...])


def _tc2_body(part_ref, acc_ref, ub_ref, u2_ref, b2_ref, out_ref):
    n = part_ref.shape[0]
    t = (acc_ref[0] + acc_ref[1])[:n]
    s = t[:, :_H]
    c = t[:, _H:_H + 1]
    agg = s / jnp.maximum(c, 1.0)
    x1 = _gelu(part_ref[...]
               + jnp.dot(agg, ub_ref[...], preferred_element_type=jnp.float32))
    x2 = _gelu(jnp.dot(x1, u2_ref[...], preferred_element_type=jnp.float32)
               + b2_ref[...])
    ss = jnp.sum(x2 * x2, axis=-1, keepdims=True)
    out_ref[...] = x2 * lax.rsqrt(jnp.maximum(ss, 1e-12))


def _splat16(vec, j):
    return lax.gather(
        vec, jnp.full((_L, 1), j, jnp.int32),
        lax.GatherDimensionNumbers(offset_dims=(), collapsed_slice_dims=(0,),
                                   start_index_map=(0,)),
        (1,), mode=lax.GatherScatterMode.PROMISE_IN_BOUNDS)


@functools.lru_cache(maxsize=None)
def _make_sc_edge(np_):
    nch = _RT // _CR
    rpt = np_ // _NS

    @functools.partial(
        pl.kernel,
        out_type=jax.ShapeDtypeStruct((_NC, np_, _HP), jnp.float32),
        mesh=plsc.VectorSubcoreMesh(core_axis_name="c", subcore_axis_name="s"),
        compiler_params=pltpu.CompilerParams(use_tc_tiling_on_sc=False),
        scratch_types=[
            pltpu.VMEM((_RT,), jnp.int32),
            pltpu.VMEM((_RT, _B), jnp.int32),
            pltpu.VMEM((_RT, _B), jnp.int32),
            pltpu.VMEM((_RT, _B), jnp.float32),
            pltpu.VMEM((_CR * _B, _HP), jnp.float32),
            pltpu.VMEM((_CR * _B, _HP), jnp.float32),
            pltpu.VMEM_SHARED((np_, _HP), jnp.float32),
            pltpu.SemaphoreType.DMA,
            pltpu.SemaphoreType.DMA,
            pltpu.SemaphoreType.DMA,
            pltpu.SemaphoreType.DMA,
            pltpu.SemaphoreType.DMA,
        ])
    def sc_edge(msgs_hbm, src_hbm, dst_hbm, w_hbm, iota_hbm, zero_hbm,
                acc_out,
                iidx_v, src_v, dst_v, w_v, rows_a, rows_b,
                acc_sh, sem_e, sem_ga, sem_gb, sem_sa, sem_sb):
        cid = lax.axis_index("c")
        sid = lax.axis_index("s")
        wid = cid * _NS + sid
        bufs = (rows_a, rows_b)
        gsems = (sem_ga, sem_gb)
        ssems = (sem_sa, sem_sb)

        r0 = sid * rpt
        pltpu.sync_copy(zero_hbm, acc_sh.at[pl.ds(r0, rpt)])
        pltpu.sync_copy(iota_hbm.at[wid], iidx_v)
        cps = [pltpu.async_copy(src_hbm.at[iidx_v], src_v, sem_e),
               pltpu.async_copy(dst_hbm.at[iidx_v], dst_v, sem_e),
               pltpu.async_copy(w_hbm.at[iidx_v], w_v, sem_e)]
        for cp in cps:
            cp.wait()
        plsc.subcore_barrier()

        def fire_gather(k):
            buf, sem = bufs[k % 2], gsems[k % 2]
            return [pltpu.async_copy(msgs_hbm.at[src_v.at[k * _CR + j]],
                                     buf.at[pl.ds(j * _B, _B)], sem)
                    for j in range(_CR)]

        def fire_scatter(k):
            buf, sem = bufs[k % 2], ssems[k % 2]
            return [pltpu.async_copy(buf.at[pl.ds(j * _B, _B)],
                                     acc_sh.at[dst_v.at[k * _CR + j]], sem,
                                     add=True)
                    for j in range(_CR)]

        def scale(k):
            buf = bufs[k % 2]

            def row_body(j, carry):
                for g in range(_B // _L):
                    w16 = w_v[k * _CR + j, pl.ds(g * _L, _L)]
                    for l in range(_L):
                        ws = _splat16(w16, l)
                        r = j * _B + g * _L + l
                        buf[r, pl.ds(0, _L)] = buf[r, pl.ds(0, _L)] * ws
                        buf[r, pl.ds(_L, _L)] = buf[r, pl.ds(_L, _L)] * ws
                return carry
            lax.fori_loop(0, _CR, row_body, 0)

        gathers = {0: fire_gather(0)}
        scatters = {}
        for k in range(nch):
            for cp in gathers.pop(k):
                cp.wait()
            if k + 1 < nch:
                if k - 1 in scatters:
                    for cp in scatters.pop(k - 1):
                        cp.wait()
                gathers[k + 1] = fire_gather(k + 1)
            scale(k)
            scatters[k] = fire_scatter(k)
        for k in sorted(scatters):
            for cp in scatters.pop(k):
                cp.wait()

        plsc.subcore_barrier()
        pltpu.sync_copy(acc_sh.at[pl.ds(r0, rpt)],
                        acc_out.at[cid, pl.ds(r0, rpt)])

    return sc_edge


def kernel(features, edges, edge_weights, params):
    n, d = features.shape
    e = edges.shape[1]
    np_ = ((n + 16 * 8 - 1) // (16 * 8)) * (16 * 8)
    ep = _NW * _RT * _B

    w1, b1 = _fold_layer(params["prepare"][0])
    w2, b2 = _fold_layer(params["prepare"][1])
    uw1, ub1 = _fold_layer(params["update"][0])
    uw2, ub2 = _fold_layer(params["update"][1])
    ua, ub = uw1[:d], uw1[d:]

    msgs, part = pl.pallas_call(
        _tc1_body,
        out_shape=[jax.ShapeDtypeStruct((n, _HP), jnp.float32),
                   jax.ShapeDtypeStruct((n, _H), jnp.float32)],
    )(features, w1, b1[None], w2, b2[None], ua, ub1[None])

    msgs_p = jnp.pad(msgs, ((0, np_ - n), (0, 0)))
    pad_idx = jnp.full((ep - e,), n, jnp.int32)
    src2d = jnp.concatenate([edges[1], pad_idx]).reshape(-1, _B)
    dst2d = jnp.concatenate([edges[0], pad_idx]).reshape(-1, _B)
    w2d = jnp.concatenate(
        [edge_weights, jnp.zeros((ep - e,), jnp.float32)]).reshape(-1, _B)
    iota = (jnp.arange(_NW, dtype=jnp.int32)[:, None] * _RT
            + jnp.arange(_RT, dtype=jnp.int32)[None, :])
    zeros = jnp.zeros((np_ // _NS, _HP), jnp.float32)
    acc = _make_sc_edge(np_)(msgs_p, src2d, dst2d, w2d, iota, zeros)

    out = pl.pallas_call(
        _tc2_body,
        out_shape=jax.ShapeDtypeStruct((n, _H), jnp.float32),
    )(part, acc, ub, uw2, ub2[None])
    return out


# speedup vs baseline: 1.9276x; 1.9276x over previous
import functools

import jax
import jax.numpy as jnp
from jax import lax
from jax.experimental import pallas as pl
from jax.experimental.pallas import tpu as pltpu
from jax.experimental.pallas import tpu_sc as plsc

_BN_EPS = 1e-3
_SQRT_HALF = 0.7071067811865476

_NC, _NS, _L = 2, 16, 16
_NW = _NC * _NS
_B = 128
_RT = 80
_CR = 5
_H = 32
_HP = 40


def _gelu(x):
    return x * 0.5 * (1.0 + lax.erf(x * _SQRT_HALF))


def _fold_layer(p):
    scale = p["gamma"] / jnp.sqrt(p["var"] + _BN_EPS)
    shift = p["beta"] - p["mean"] * scale
    w = scale[:, None] * p["W"]
    b = shift @ p["W"] + p["b"]
    return w, b


def _tc1_body(f_ref, w1_ref, b1_ref, w2_ref, b2_ref, ua_ref, u1_ref,
              msgs_ref, part_ref):
    x = f_ref[...]
    n = x.shape[0]
    h1 = _gelu(jnp.dot(x, w1_ref[...], preferred_element_type=jnp.float32)
               + b1_ref[...])
    m = _gelu(jnp.dot(h1, w2_ref[...], preferred_element_type=jnp.float32)
              + b2_ref[...])
    col = lax.broadcasted_iota(jnp.int32, (n, _HP - _H), 1)
    tail = jnp.where(col == 0, 1.0, 0.0).astype(jnp.float32)
    msgs_ref[...] = jnp.concatenate([m, tail], axis=1)
    part_ref[...] = (jnp.dot(x, ua_ref[...], preferred_element_type=jnp.float32)
                     + u1_ref[...])


def _tc2_body(part_ref, acc_ref, ub_ref, u2_ref, b2_ref, out_ref):
    n = part_ref.shape[0]
    t = (acc_ref[0] + acc_ref[1])[:n]
    s = t[:, :_H]
    c = t[:, _H:_H + 1]
    agg = s / jnp.maximum(c, 1.0)
    x1 = _gelu(part_ref[...]
               + jnp.dot(agg, ub_ref[...], preferred_element_type=jnp.float32))
    x2 = _gelu(jnp.dot(x1, u2_ref[...], preferred_element_type=jnp.float32)
               + b2_ref[...])
    ss = jnp.sum(x2 * x2, axis=-1, keepdims=True)
    out_ref[...] = x2 * lax.rsqrt(jnp.maximum(ss, 1e-12))


def _splat16(vec, j):
    return lax.gather(
        vec, jnp.full((_L, 1), j, jnp.int32),
        lax.GatherDimensionNumbers(offset_dims=(), collapsed_slice_dims=(0,),
                                   start_index_map=(0,)),
        (1,), mode=lax.GatherScatterMode.PROMISE_IN_BOUNDS)


@functools.lru_cache(maxsize=None)
def _make_sc_edge(np_):
    nch = _RT // _CR
    rpt = np_ // _NS

    @functools.partial(
        pl.kernel,
        out_type=jax.ShapeDtypeStruct((_NC, np_, _HP), jnp.float32),
        mesh=plsc.VectorSubcoreMesh(core_axis_name="c", subcore_axis_name="s"),
        compiler_params=pltpu.CompilerParams(use_tc_tiling_on_sc=False),
        scratch_types=[
            pltpu.VMEM((_RT,), jnp.int32),
            pltpu.VMEM((_RT, _B), jnp.int32),
            pltpu.VMEM((_RT, _B), jnp.int32),
            pltpu.VMEM((_RT, _B), jnp.float32),
            pltpu.VMEM((_CR * _B, _HP), jnp.float32),
            pltpu.VMEM((_CR * _B, _HP), jnp.float32),
            pltpu.VMEM_SHARED((np_, _HP), jnp.float32),
            pltpu.SemaphoreType.DMA,
            pltpu.SemaphoreType.DMA,
            pltpu.SemaphoreType.DMA,
            pltpu.SemaphoreType.DMA,
            pltpu.SemaphoreType.DMA,
        ])
    def sc_edge(msgs_hbm, src_hbm, dst_hbm, w_hbm, iota_hbm, zero_hbm,
                acc_out,
                iidx_v, src_v, dst_v, w_v, rows_a, rows_b,
                acc_sh, sem_e, sem_ga, sem_gb, sem_sa, sem_sb):
        cid = lax.axis_index("c")
        sid = lax.axis_index("s")
        wid = cid * _NS + sid
        bufs = (rows_a, rows_b)
        gsems = (sem_ga, sem_gb)
        ssems = (sem_sa, sem_sb)

        r0 = sid * rpt
        pltpu.sync_copy(zero_hbm, acc_sh.at[pl.ds(r0, rpt)])
        pltpu.sync_copy(iota_hbm.at[wid], iidx_v)
        cps = [pltpu.async_copy(src_hbm.at[iidx_v], src_v, sem_e),
               pltpu.async_copy(dst_hbm.at[iidx_v], dst_v, sem_e),
               pltpu.async_copy(w_hbm.at[iidx_v], w_v, sem_e)]
        for cp in cps:
            cp.wait()
        plsc.subcore_barrier()

        def fire_gather(k):
            buf, sem = bufs[k % 2], gsems[k % 2]
            return [pltpu.async_copy(msgs_hbm.at[src_v.at[k * _CR + j]],
                                     buf.at[pl.ds(j * _B, _B)], sem)
                    for j in range(_CR)]

        def fire_scatter(k):
            buf, sem = bufs[k % 2], ssems[k % 2]
            return [pltpu.async_copy(buf.at[pl.ds(j * _B, _B)],
                                     acc_sh.at[dst_v.at[k * _CR + j]], sem,
                                     add=True)
                    for j in range(_CR)]

        def scale(k):
            buf = bufs[k % 2]

            def row_body(j, carry):
                for g in range(_B // _L):
                    w16 = w_v[k * _CR + j, pl.ds(g * _L, _L)]
                    for l in range(_L):
                        ws = _splat16(w16, l)
                        r = j * _B + g * _L + l
                        buf[r, pl.ds(0, _L)] = buf[r, pl.ds(0, _L)] * ws
                        buf[r, pl.ds(_L, _L)] = buf[r, pl.ds(_L, _L)] * ws
                return carry
            lax.fori_loop(0, _CR, row_body, 0)

        gathers = {0: fire_gather(0)}
        scatters = {}
        for k in range(nch):
            for cp in gathers.pop(k):
                cp.wait()
            if k + 1 < nch:
                if k - 1 in scatters:
                    for cp in scatters.pop(k - 1):
                        cp.wait()
                gathers[k + 1] = fire_gather(k + 1)
            scale(k)
            scatters[k] = fire_scatter(k)
        for k in sorted(scatters):
            for cp in scatters.pop(k):
                cp.wait()

        plsc.subcore_barrier()
        pltpu.sync_copy(acc_sh.at[pl.ds(r0, rpt)],
                        acc_out.at[cid, pl.ds(r0, rpt)])

    return sc_edge


def kernel(features, edges, edge_weights, params):
    n, d = features.shape
    e = edges.shape[1]
    np_ = ((n + 16 * 8 - 1) // (16 * 8)) * (16 * 8)
    ep = _NW * _RT * _B

    w1, b1 = _fold_layer(params["prepare"][0])
    w2, b2 = _fold_layer(params["prepare"][1])
    uw1, ub1 = _fold_layer(params["update"][0])
    uw2, ub2 = _fold_layer(params["update"][1])
    ua, ub = uw1[:d], uw1[d:]

    msgs, part = pl.pallas_call(
        _tc1_body,
        out_shape=[jax.ShapeDtypeStruct((n, _HP), jnp.float32),
                   jax.ShapeDtypeStruct((n, _H), jnp.float32)],
    )(features, w1, b1[None], w2, b2[None], ua, ub1[None])

    msgs_p = jnp.pad(msgs, ((0, np_ - n), (0, 0)))
    pad_idx = n + jnp.arange(ep - e, dtype=jnp.int32) % (np_ - n)
    src2d = jnp.concatenate([edges[1], pad_idx]).reshape(-1, _B)
    dst2d = jnp.concatenate([edges[0], pad_idx]).reshape(-1, _B)
    w2d = jnp.concatenate(
        [edge_weights, jnp.zeros((ep - e,), jnp.float32)]).reshape(-1, _B)
    iota = (jnp.arange(_NW, dtype=jnp.int32)[:, None] * _RT
            + jnp.arange(_RT, dtype=jnp.int32)[None, :])
    zeros = jnp.zeros((np_ // _NS, _HP), jnp.float32)
    acc = _make_sc_edge(np_)(msgs_p, src2d, dst2d, w2d, iota, zeros)

    out = pl.pallas_call(
        _tc2_body,
        out_shape=jax.ShapeDtypeStruct((n, _H), jnp.float32),
    )(part, acc, ub, uw2, ub2[None])
    return out


# speedup vs baseline: 2.1625x; 1.1219x over previous
import functools

import jax
import jax.numpy as jnp
from jax import lax
from jax.experimental import pallas as pl
from jax.experimental.pallas import tpu as pltpu
from jax.experimental.pallas import tpu_sc as plsc

_BN_EPS = 1e-3
_SQRT_HALF = 0.7071067811865476

_NC, _NS, _L = 2, 16, 16
_NW = _NC * _NS
_B = 128
_RT = 80
_CR = 2
_NB = 4
_H = 32
_HP = 40


def _gelu(x):
    return x * 0.5 * (1.0 + lax.erf(x * _SQRT_HALF))


def _fold_layer(p):
    scale = p["gamma"] / jnp.sqrt(p["var"] + _BN_EPS)
    shift = p["beta"] - p["mean"] * scale
    w = scale[:, None] * p["W"]
    b = shift @ p["W"] + p["b"]
    return w, b


def _tc1_body(f_ref, w1_ref, b1_ref, w2_ref, b2_ref, ua_ref, u1_ref,
              msgs_ref, part_ref):
    x = f_ref[...]
    n = x.shape[0]
    np_ = msgs_ref.shape[0]
    h1 = _gelu(jnp.dot(x, w1_ref[...], preferred_element_type=jnp.float32)
               + b1_ref[...])
    m = _gelu(jnp.dot(h1, w2_ref[...], preferred_element_type=jnp.float32)
              + b2_ref[...])
    col = lax.broadcasted_iota(jnp.int32, (n, _HP - _H), 1)
    tail = jnp.where(col == 0, 1.0, 0.0).astype(jnp.float32)
    msgs_ref[pl.ds(0, n), :] = jnp.concatenate([m, tail], axis=1)
    msgs_ref[pl.ds(n, np_ - n), :] = jnp.zeros((np_ - n, _HP), jnp.float32)
    part_ref[...] = (jnp.dot(x, ua_ref[...], preferred_element_type=jnp.float32)
                     + u1_ref[...])


def _tc2_body(part_ref, acc_ref, ub_ref, u2_ref, b2_ref, out_ref):
    n = part_ref.shape[0]
    t = (acc_ref[0] + acc_ref[1])[:n]
    s = t[:, :_H]
    c = t[:, _H:_H + 1]
    agg = s / jnp.maximum(c, 1.0)
    x1 = _gelu(part_ref[...]
               + jnp.dot(agg, ub_ref[...], preferred_element_type=jnp.float32))
    x2 = _gelu(jnp.dot(x1, u2_ref[...], preferred_element_type=jnp.float32)
               + b2_ref[...])
    ss = jnp.sum(x2 * x2, axis=-1, keepdims=True)
    out_ref[...] = x2 * lax.rsqrt(jnp.maximum(ss, 1e-12))


def _splat16(vec, j):
    return lax.gather(
        vec, jnp.full((_L, 1), j, jnp.int32),
        lax.GatherDimensionNumbers(offset_dims=(), collapsed_slice_dims=(0,),
                                   start_index_map=(0,)),
        (1,), mode=lax.GatherScatterMode.PROMISE_IN_BOUNDS)


@functools.lru_cache(maxsize=None)
def _make_sc_edge(np_):
    nch = _RT // _CR
    rpt = np_ // _NS

    @functools.partial(
        pl.kernel,
        out_type=jax.ShapeDtypeStruct((_NC, np_, _HP), jnp.float32),
        mesh=plsc.VectorSubcoreMesh(core_axis_name="c", subcore_axis_name="s"),
        compiler_params=pltpu.CompilerParams(use_tc_tiling_on_sc=False),
        scratch_types=[
            pltpu.VMEM((_RT,), jnp.int32),
            pltpu.VMEM((_RT, _B), jnp.int32),
            pltpu.VMEM((_RT, _B), jnp.int32),
            pltpu.VMEM((_RT, _B), jnp.float32),
        ] + [pltpu.VMEM((_CR * _B, _HP), jnp.float32)] * _NB
          + [pltpu.VMEM_SHARED((np_, _HP), jnp.float32)]
          + [pltpu.SemaphoreType.DMA] * (1 + 2 * _NB))
    def sc_edge(msgs_hbm, src_hbm, dst_hbm, w_hbm, iota_hbm, zero_hbm,
                acc_out,
                iidx_v, src_v, dst_v, w_v, *rest):
        bufs = rest[:_NB]
        acc_sh = rest[_NB]
        sem_e = rest[_NB + 1]
        gsems = rest[_NB + 2:_NB + 2 + _NB]
        ssems = rest[_NB + 2 + _NB:]
        cid = lax.axis_index("c")
        sid = lax.axis_index("s")
        wid = cid * _NS + sid

        r0 = sid * rpt
        pltpu.sync_copy(zero_hbm, acc_sh.at[pl.ds(r0, rpt)])
        pltpu.sync_copy(iota_hbm.at[wid], iidx_v)
        cps = [pltpu.async_copy(src_hbm.at[iidx_v], src_v, sem_e),
               pltpu.async_copy(dst_hbm.at[iidx_v], dst_v, sem_e),
               pltpu.async_copy(w_hbm.at[iidx_v], w_v, sem_e)]
        for cp in cps:
            cp.wait()
        plsc.subcore_barrier()

        def fire_gather(k, b):
            buf, sem = bufs[b], gsems[b]
            for j in range(_CR):
                pltpu.async_copy(msgs_hbm.at[src_v.at[k * _CR + j]],
                                 buf.at[pl.ds(j * _B, _B)], sem)

        def wait_gather(b):
            buf, sem = bufs[b], gsems[b]
            for j in range(_CR):
                pltpu.make_async_copy(msgs_hbm.at[src_v.at[0]],
                                      buf.at[pl.ds(j * _B, _B)], sem).wait()

        def fire_scatter(k, b):
            buf, sem = bufs[b], ssems[b]
            for j in range(_CR):
                pltpu.async_copy(buf.at[pl.ds(j * _B, _B)],
                                 acc_sh.at[dst_v.at[k * _CR + j]], sem,
                                 add=True)

        def wait_scatter(b):
            buf, sem = bufs[b], ssems[b]
            for j in range(_CR):
                pltpu.make_async_copy(buf.at[pl.ds(j * _B, _B)],
                                      acc_sh.at[dst_v.at[0]], sem).wait()

        def scale(k, b):
            buf = bufs[b]

            def grp_body(t, carry):
                w16 = w_v[k * _CR + t // (_B // _L),
                          pl.ds((t % (_B // _L)) * _L, _L)]
                for l in range(_L):
                    ws = _splat16(w16, l)
                    r = t * _L + l
                    buf[r, pl.ds(0, _L)] = buf[r, pl.ds(0, _L)] * ws
                    buf[r, pl.ds(_L, _L)] = buf[r, pl.ds(_L, _L)] * ws
                return carry
            lax.fori_loop(0, _CR * _B // _L, grp_body, 0)

        nouter = nch // _NB
        fire_gather(0, 0)
        fire_gather(1, 1)
        fire_gather(2, 2)
        wait_gather(0); scale(0, 0); fire_scatter(0, 0)
        fire_gather(3, 3)
        wait_gather(1); scale(1, 1); fire_scatter(1, 1)
        wait_scatter(0); fire_gather(4, 0)
        wait_gather(2); scale(2, 2); fire_scatter(2, 2)
        wait_scatter(1); fire_gather(5, 1)
        wait_gather(3); scale(3, 3); fire_scatter(3, 3)

        def outer_body(o, carry):
            for i in range(_NB):
                k = o * _NB + i
                wait_scatter((i + 2) % _NB)
                fire_gather(k + 2, (i + 2) % _NB)
                wait_gather(i)
                scale(k, i)
                fire_scatter(k, i)
            return carry
        lax.fori_loop(1, nouter - 1, outer_body, 0)

        k0 = (nouter - 1) * _NB
        wait_scatter(2); fire_gather(k0 + 2, 2)
        wait_gather(0); scale(k0 + 0, 0); fire_scatter(k0 + 0, 0)
        wait_scatter(3); fire_gather(k0 + 3, 3)
        wait_gather(1); scale(k0 + 1, 1); fire_scatter(k0 + 1, 1)
        wait_scatter(0)
        wait_gather(2); scale(k0 + 2, 2); fire_scatter(k0 + 2, 2)
        wait_scatter(1)
        wait_gather(3); scale(k0 + 3, 3); fire_scatter(k0 + 3, 3)
        wait_scatter(2)
        wait_scatter(3)

        plsc.subcore_barrier()
        pltpu.sync_copy(acc_sh.at[pl.ds(r0, rpt)],
                        acc_out.at[cid, pl.ds(r0, rpt)])

    return sc_edge


def kernel(features, edges, edge_weights, params):
    n, d = features.shape
    e = edges.shape[1]
    np_ = ((n + 16 * 8 - 1) // (16 * 8)) * (16 * 8)
    ep = _NW * _RT * _B

    w1, b1 = _fold_layer(params["prepare"][0])
    w2, b2 = _fold_layer(params["prepare"][1])
    uw1, ub1 = _fold_layer(params["update"][0])
    uw2, ub2 = _fold_layer(params["update"][1])
    ua, ub = uw1[:d], uw1[d:]

    msgs_p, part = pl.pallas_call(
        _tc1_body,
        out_shape=[jax.ShapeDtypeStruct((np_, _HP), jnp.float32),
                   jax.ShapeDtypeStruct((n, _H), jnp.float32)],
    )(features, w1, b1[None], w2, b2[None], ua, ub1[None])

    pad_idx = n + jnp.arange(ep - e, dtype=jnp.int32) % (np_ - n)
    src2d = jnp.concatenate([edges[1], pad_idx]).reshape(-1, _B)
    dst2d = jnp.concatenate([edges[0], pad_idx]).reshape(-1, _B)
    w2d = jnp.concatenate(
        [edge_weights, jnp.zeros((ep - e,), jnp.float32)]).reshape(-1, _B)
    iota = (jnp.arange(_NW, dtype=jnp.int32)[:, None] * _RT
            + jnp.arange(_RT, dtype=jnp.int32)[None, :])
    zeros = jnp.zeros((np_ // _NS, _HP), jnp.float32)
    acc = _make_sc_edge(np_)(msgs_p, src2d, dst2d, w2d, iota, zeros)

    out = pl.pallas_call(
        _tc2_body,
        out_shape=jax.ShapeDtypeStruct((n, _H), jnp.float32),
    )(part, acc, ub, uw2, ub2[None])
    return out


# speedup vs baseline: 2.1724x; 1.0046x over previous
import functools

import jax
import jax.numpy as jnp
from jax import lax
from jax.experimental import pallas as pl
from jax.experimental.pallas import tpu as pltpu
from jax.experimental.pallas import tpu_sc as plsc

_BN_EPS = 1e-3
_SQRT_HALF = 0.7071067811865476

_NC, _NS, _L = 2, 16, 16
_NW = _NC * _NS
_B = 128
_RT = 80
_CR = 2
_NB = 4
_H = 32
_HP = 40


def _gelu(x):
    return x * 0.5 * (1.0 + lax.erf(x * _SQRT_HALF))


def _fold_layer(p):
    scale = p["gamma"] / jnp.sqrt(p["var"] + _BN_EPS)
    shift = p["beta"] - p["mean"] * scale
    w = scale[:, None] * p["W"]
    b = shift @ p["W"] + p["b"]
    return w, b


def _tc1_body(f_ref, w1_ref, b1_ref, w2_ref, b2_ref, ua_ref, u1_ref,
              msgs_ref, part_ref):
    x = f_ref[...]
    n = x.shape[0]
    h1 = _gelu(jnp.dot(x, w1_ref[...], preferred_element_type=jnp.float32)
               + b1_ref[...])
    m = _gelu(jnp.dot(h1, w2_ref[...], preferred_element_type=jnp.float32)
              + b2_ref[...])
    col = lax.broadcasted_iota(jnp.int32, (n, _HP - _H), 1)
    tail = jnp.where(col == 0, 1.0, 0.0).astype(jnp.float32)
    msgs_ref[...] = jnp.concatenate([m, tail], axis=1)
    part_ref[...] = (jnp.dot(x, ua_ref[...], preferred_element_type=jnp.float32)
                     + u1_ref[...])


def _tc2_body(part_ref, acc_ref, ub_ref, u2_ref, b2_ref, out_ref):
    t = acc_ref[0] + acc_ref[1]
    s = t[:, :_H]
    c = t[:, _H:_H + 1]
    agg = s / jnp.maximum(c, 1.0)
    x1 = _gelu(part_ref[...]
               + jnp.dot(agg, ub_ref[...], preferred_element_type=jnp.float32))
    x2 = _gelu(jnp.dot(x1, u2_ref[...], preferred_element_type=jnp.float32)
               + b2_ref[...])
    ss = jnp.sum(x2 * x2, axis=-1, keepdims=True)
    out_ref[...] = x2 * lax.rsqrt(jnp.maximum(ss, 1e-12))


def _splat16(vec, j):
    return lax.gather(
        vec, jnp.full((_L, 1), j, jnp.int32),
        lax.GatherDimensionNumbers(offset_dims=(), collapsed_slice_dims=(0,),
                                   start_index_map=(0,)),
        (1,), mode=lax.GatherScatterMode.PROMISE_IN_BOUNDS)


@functools.lru_cache(maxsize=None)
def _make_sc_edge(np_):
    nch = _RT // _CR
    rpt = np_ // _NS

    @functools.partial(
        pl.kernel,
        out_type=jax.ShapeDtypeStruct((_NC, np_, _HP), jnp.float32),
        mesh=plsc.VectorSubcoreMesh(core_axis_name="c", subcore_axis_name="s"),
        compiler_params=pltpu.CompilerParams(use_tc_tiling_on_sc=False),
        scratch_types=[
            pltpu.VMEM((_RT,), jnp.int32),
            pltpu.VMEM((_RT, _B), jnp.int32),
            pltpu.VMEM((_RT, _B), jnp.int32),
            pltpu.VMEM((_RT, _B), jnp.float32),
        ] + [pltpu.VMEM((_CR * _B, _HP), jnp.float32)] * _NB
          + [pltpu.VMEM_SHARED((np_, _HP), jnp.float32)]
          + [pltpu.SemaphoreType.DMA] * (1 + 2 * _NB))
    def sc_edge(msgs_hbm, src_hbm, dst_hbm, w_hbm, iota_hbm, zero_hbm,
                acc_out,
                iidx_v, src_v, dst_v, w_v, *rest):
        bufs = rest[:_NB]
        acc_sh = rest[_NB]
        sem_e = rest[_NB + 1]
        gsems = rest[_NB + 2:_NB + 2 + _NB]
        ssems = rest[_NB + 2 + _NB:]
        cid = lax.axis_index("c")
        sid = lax.axis_index("s")
        wid = cid * _NS + sid

        r0 = sid * rpt
        pltpu.sync_copy(zero_hbm, acc_sh.at[pl.ds(r0, rpt)])
        pltpu.sync_copy(iota_hbm.at[wid], iidx_v)
        cps = [pltpu.async_copy(src_hbm.at[iidx_v], src_v, sem_e),
               pltpu.async_copy(dst_hbm.at[iidx_v], dst_v, sem_e),
               pltpu.async_copy(w_hbm.at[iidx_v], w_v, sem_e)]
        for cp in cps:
            cp.wait()
        plsc.subcore_barrier()

        def fire_gather(k, b):
            buf, sem = bufs[b], gsems[b]
            for j in range(_CR):
                pltpu.async_copy(msgs_hbm.at[src_v.at[k * _CR + j]],
                                 buf.at[pl.ds(j * _B, _B)], sem)

        def wait_gather(b):
            buf, sem = bufs[b], gsems[b]
            for j in range(_CR):
                pltpu.make_async_copy(msgs_hbm.at[src_v.at[0]],
                                      buf.at[pl.ds(j * _B, _B)], sem).wait()

        def fire_scatter(k, b):
            buf, sem = bufs[b], ssems[b]
            for j in range(_CR):
                pltpu.async_copy(buf.at[pl.ds(j * _B, _B)],
                                 acc_sh.at[dst_v.at[k * _CR + j]], sem,
                                 add=True)

        def wait_scatter(b):
            buf, sem = bufs[b], ssems[b]
            for j in range(_CR):
                pltpu.make_async_copy(buf.at[pl.ds(j * _B, _B)],
                                      acc_sh.at[dst_v.at[0]], sem).wait()

        def scale(k, b):
            buf = bufs[b]

            def grp_body(t, carry):
                w16 = w_v[k * _CR + t // (_B // _L),
                          pl.ds((t % (_B // _L)) * _L, _L)]
                for l in range(_L):
                    ws = _splat16(w16, l)
                    r = t * _L + l
                    buf[r, pl.ds(0, _L)] = buf[r, pl.ds(0, _L)] * ws
                    buf[r, pl.ds(_L, _L)] = buf[r, pl.ds(_L, _L)] * ws
                return carry
            lax.fori_loop(0, _CR * _B // _L, grp_body, 0)

        nouter = nch // _NB
        fire_gather(0, 0)
        fire_gather(1, 1)
        fire_gather(2, 2)
        wait_gather(0); scale(0, 0); fire_scatter(0, 0)
        fire_gather(3, 3)
        wait_gather(1); scale(1, 1); fire_scatter(1, 1)
        wait_scatter(0); fire_gather(4, 0)
        wait_gather(2); scale(2, 2); fire_scatter(2, 2)
        wait_scatter(1); fire_gather(5, 1)
        wait_gather(3); scale(3, 3); fire_scatter(3, 3)

        def outer_body(o, carry):
            for i in range(_NB):
                k = o * _NB + i
                wait_scatter((i + 2) % _NB)
                fire_gather(k + 2, (i + 2) % _NB)
                wait_gather(i)
                scale(k, i)
                fire_scatter(k, i)
            return carry
        lax.fori_loop(1, nouter - 1, outer_body, 0)

        k0 = (nouter - 1) * _NB
        wait_scatter(2); fire_gather(k0 + 2, 2)
        wait_gather(0); scale(k0 + 0, 0); fire_scatter(k0 + 0, 0)
        wait_scatter(3); fire_gather(k0 + 3, 3)
        wait_gather(1); scale(k0 + 1, 1); fire_scatter(k0 + 1, 1)
        wait_scatter(0)
        wait_gather(2); scale(k0 + 2, 2); fire_scatter(k0 + 2, 2)
        wait_scatter(1)
        wait_gather(3); scale(k0 + 3, 3); fire_scatter(k0 + 3, 3)
        wait_scatter(2)
        wait_scatter(3)

        plsc.subcore_barrier()
        pltpu.sync_copy(acc_sh.at[pl.ds(r0, rpt)],
                        acc_out.at[cid, pl.ds(r0, rpt)])

    return sc_edge


def kernel(features, edges, edge_weights, params):
    n, d = features.shape
    e = edges.shape[1]
    np_ = ((n + 16 * 8 - 1) // (16 * 8)) * (16 * 8)
    ep = _NW * _RT * _B

    w1, b1 = _fold_layer(params["prepare"][0])
    w2, b2 = _fold_layer(params["prepare"][1])
    uw1, ub1 = _fold_layer(params["update"][0])
    uw2, ub2 = _fold_layer(params["update"][1])
    ua, ub = uw1[:d], uw1[d:]

    grid = 5
    bn = n // grid
    wspec = [pl.BlockSpec(ws.shape, lambda i: (0,) * ws.ndim)
             for ws in (w1, b1[None], w2, b2[None], ua, ub1[None])]
    msgs_p, part = pl.pallas_call(
        _tc1_body,
        grid=(grid,),
        in_specs=[pl.BlockSpec((bn, d), lambda i: (i, 0))] + wspec,
        out_specs=[pl.BlockSpec((bn, _HP), lambda i: (i, 0)),
                   pl.BlockSpec((bn, _H), lambda i: (i, 0))],
        out_shape=[jax.ShapeDtypeStruct((n, _HP), jnp.float32),
                   jax.ShapeDtypeStruct((n, _H), jnp.float32)],
    )(features, w1, b1[None], w2, b2[None], ua, ub1[None])

    cnt = ep - e
    pad_src = jnp.arange(cnt, dtype=jnp.int32) % n
    pad_dst = n + jnp.arange(cnt, dtype=jnp.int32) % (np_ - n)
    src2d = jnp.concatenate([edges[1], pad_src]).reshape(-1, _B)
    dst2d = jnp.concatenate([edges[0], pad_dst]).reshape(-1, _B)
    w2d = jnp.concatenate(
        [edge_weights, jnp.zeros((cnt,), jnp.float32)]).reshape(-1, _B)
    iota = (jnp.arange(_NW, dtype=jnp.int32)[:, None] * _RT
            + jnp.arange(_RT, dtype=jnp.int32)[None, :])
    zeros = jnp.zeros((np_ // _NS, _HP), jnp.float32)
    acc = _make_sc_edge(np_)(msgs_p, src2d, dst2d, w2d, iota, zeros)

    wspec2 = [pl.BlockSpec(ws.shape, lambda i: (0,) * ws.ndim)
              for ws in (ub, uw2, ub2[None])]
    out = pl.pallas_call(
        _tc2_body,
        grid=(grid,),
        in_specs=[pl.BlockSpec((bn, _H), lambda i: (i, 0)),
                  pl.BlockSpec((_NC, bn, _HP), lambda i: (0, i, 0))] + wspec2,
        out_specs=pl.BlockSpec((bn, _H), lambda i: (i, 0)),
        out_shape=jax.ShapeDtypeStruct((n, _H), jnp.float32),
    )(part, acc, ub, uw2, ub2[None])
    return out


# speedup vs baseline: 2.1781x; 1.0026x over previous
import functools

import jax
import jax.numpy as jnp
from jax import lax
from jax.experimental import pallas as pl
from jax.experimental.pallas import tpu as pltpu
from jax.experimental.pallas import tpu_sc as plsc

_BN_EPS = 1e-3
_SQRT_HALF = 0.7071067811865476

_NC, _NS, _L = 2, 16, 16
_NW = _NC * _NS
_B = 256
_RT = 40
_CR = 1
_NB = 4
_H = 32
_HP = 40


def _gelu(x):
    return x * 0.5 * (1.0 + lax.erf(x * _SQRT_HALF))


def _fold_layer(p):
    scale = p["gamma"] / jnp.sqrt(p["var"] + _BN_EPS)
    shift = p["beta"] - p["mean"] * scale
    w = scale[:, None] * p["W"]
    b = shift @ p["W"] + p["b"]
    return w, b


def _tc1_body(f_ref, w1_ref, b1_ref, w2_ref, b2_ref, ua_ref, u1_ref,
              msgs_ref, part_ref):
    x = f_ref[...]
    n = x.shape[0]
    h1 = _gelu(jnp.dot(x, w1_ref[...], preferred_element_type=jnp.float32)
               + b1_ref[...])
    m = _gelu(jnp.dot(h1, w2_ref[...], preferred_element_type=jnp.float32)
              + b2_ref[...])
    col = lax.broadcasted_iota(jnp.int32, (n, _HP - _H), 1)
    tail = jnp.where(col == 0, 1.0, 0.0).astype(jnp.float32)
    msgs_ref[...] = jnp.concatenate([m, tail], axis=1)
    part_ref[...] = (jnp.dot(x, ua_ref[...], preferred_element_type=jnp.float32)
                     + u1_ref[...])


def _tc2_body(part_ref, acc_ref, ub_ref, u2_ref, b2_ref, out_ref):
    t = acc_ref[0] + acc_ref[1]
    s = t[:, :_H]
    c = t[:, _H:_H + 1]
    agg = s / jnp.maximum(c, 1.0)
    x1 = _gelu(part_ref[...]
               + jnp.dot(agg, ub_ref[...], preferred_element_type=jnp.float32))
    x2 = _gelu(jnp.dot(x1, u2_ref[...], preferred_element_type=jnp.float32)
               + b2_ref[...])
    ss = jnp.sum(x2 * x2, axis=-1, keepdims=True)
    out_ref[...] = x2 * lax.rsqrt(jnp.maximum(ss, 1e-12))


def _splat16(vec, j):
    return lax.gather(
        vec, jnp.full((_L, 1), j, jnp.int32),
        lax.GatherDimensionNumbers(offset_dims=(), collapsed_slice_dims=(0,),
                                   start_index_map=(0,)),
        (1,), mode=lax.GatherScatterMode.PROMISE_IN_BOUNDS)


@functools.lru_cache(maxsize=None)
def _make_sc_edge(np_):
    nch = _RT // _CR
    rpt = np_ // _NS

    @functools.partial(
        pl.kernel,
        out_type=jax.ShapeDtypeStruct((_NC, np_, _HP), jnp.float32),
        mesh=plsc.VectorSubcoreMesh(core_axis_name="c", subcore_axis_name="s"),
        compiler_params=pltpu.CompilerParams(use_tc_tiling_on_sc=False),
        scratch_types=[
            pltpu.VMEM((_RT,), jnp.int32),
            pltpu.VMEM((_RT, _B), jnp.int32),
            pltpu.VMEM((_RT, _B), jnp.int32),
            pltpu.VMEM((_RT, _B), jnp.float32),
        ] + [pltpu.VMEM((_CR * _B, _HP), jnp.float32)] * _NB
          + [pltpu.VMEM_SHARED((np_, _HP), jnp.float32)]
          + [pltpu.SemaphoreType.DMA] * (1 + 2 * _NB))
    def sc_edge(msgs_hbm, src_hbm, dst_hbm, w_hbm, iota_hbm, zero_hbm,
                acc_out,
                iidx_v, src_v, dst_v, w_v, *rest):
        bufs = rest[:_NB]
        acc_sh = rest[_NB]
        sem_e = rest[_NB + 1]
        gsems = rest[_NB + 2:_NB + 2 + _NB]
        ssems = rest[_NB + 2 + _NB:]
        cid = lax.axis_index("c")
        sid = lax.axis_index("s")
        wid = cid * _NS + sid

        r0 = sid * rpt
        pltpu.sync_copy(zero_hbm, acc_sh.at[pl.ds(r0, rpt)])
        pltpu.sync_copy(iota_hbm.at[wid], iidx_v)
        cps = [pltpu.async_copy(src_hbm.at[iidx_v], src_v, sem_e),
               pltpu.async_copy(dst_hbm.at[iidx_v], dst_v, sem_e),
               pltpu.async_copy(w_hbm.at[iidx_v], w_v, sem_e)]
        for cp in cps:
            cp.wait()
        plsc.subcore_barrier()

        def fire_gather(k, b):
            buf, sem = bufs[b], gsems[b]
            for j in range(_CR):
                pltpu.async_copy(msgs_hbm.at[src_v.at[k * _CR + j]],
                                 buf.at[pl.ds(j * _B, _B)], sem)

        def wait_gather(b):
            buf, sem = bufs[b], gsems[b]
            for j in range(_CR):
                pltpu.make_async_copy(msgs_hbm.at[src_v.at[0]],
                                      buf.at[pl.ds(j * _B, _B)], sem).wait()

        def fire_scatter(k, b):
            buf, sem = bufs[b], ssems[b]
            for j in range(_CR):
                pltpu.async_copy(buf.at[pl.ds(j * _B, _B)],
                                 acc_sh.at[dst_v.at[k * _CR + j]], sem,
                                 add=True)

        def wait_scatter(b):
            buf, sem = bufs[b], ssems[b]
            for j in range(_CR):
                pltpu.make_async_copy(buf.at[pl.ds(j * _B, _B)],
                                      acc_sh.at[dst_v.at[0]], sem).wait()

        def scale(k, b):
            buf = bufs[b]

            def grp_body(t, carry):
                w16 = w_v[k * _CR + t // (_B // _L),
                          pl.ds((t % (_B // _L)) * _L, _L)]
                for l in range(_L):
                    ws = _splat16(w16, l)
                    r = t * _L + l
                    buf[r, pl.ds(0, _L)] = buf[r, pl.ds(0, _L)] * ws
                    buf[r, pl.ds(_L, _L)] = buf[r, pl.ds(_L, _L)] * ws
                return carry
            lax.fori_loop(0, _CR * _B // _L, grp_body, 0)

        nouter = nch // _NB
        fire_gather(0, 0)
        fire_gather(1, 1)
        fire_gather(2, 2)
        wait_gather(0); scale(0, 0); fire_scatter(0, 0)
        fire_gather(3, 3)
        wait_gather(1); scale(1, 1); fire_scatter(1, 1)
        wait_scatter(0); fire_gather(4, 0)
        wait_gather(2); scale(2, 2); fire_scatter(2, 2)
        wait_scatter(1); fire_gather(5, 1)
        wait_gather(3); scale(3, 3); fire_scatter(3, 3)

        def outer_body(o, carry):
            for i in range(_NB):
                k = o * _NB + i
                wait_scatter((i + 2) % _NB)
                fire_gather(k + 2, (i + 2) % _NB)
                wait_gather(i)
                scale(k, i)
                fire_scatter(k, i)
            return carry
        lax.fori_loop(1, nouter - 1, outer_body, 0)

        k0 = (nouter - 1) * _NB
        wait_scatter(2); fire_gather(k0 + 2, 2)
        wait_gather(0); scale(k0 + 0, 0); fire_scatter(k0 + 0, 0)
        wait_scatter(3); fire_gather(k0 + 3, 3)
        wait_gather(1); scale(k0 + 1, 1); fire_scatter(k0 + 1, 1)
        wait_scatter(0)
        wait_gather(2); scale(k0 + 2, 2); fire_scatter(k0 + 2, 2)
        wait_scatter(1)
        wait_gather(3); scale(k0 + 3, 3); fire_scatter(k0 + 3, 3)
        wait_scatter(2)
        wait_scatter(3)

        plsc.subcore_barrier()
        pltpu.sync_copy(acc_sh.at[pl.ds(r0, rpt)],
                        acc_out.at[cid, pl.ds(r0, rpt)])

    return sc_edge


def kernel(features, edges, edge_weights, params):
    n, d = features.shape
    e = edges.shape[1]
    np_ = ((n + 16 * 8 - 1) // (16 * 8)) * (16 * 8)
    ep = _NW * _RT * _B

    w1, b1 = _fold_layer(params["prepare"][0])
    w2, b2 = _fold_layer(params["prepare"][1])
    uw1, ub1 = _fold_layer(params["update"][0])
    uw2, ub2 = _fold_layer(params["update"][1])
    ua, ub = uw1[:d], uw1[d:]

    grid = 5
    bn = n // grid
    wspec = [pl.BlockSpec(ws.shape, lambda i: (0,) * ws.ndim)
             for ws in (w1, b1[None], w2, b2[None], ua, ub1[None])]
    msgs_p, part = pl.pallas_call(
        _tc1_body,
        grid=(grid,),
        in_specs=[pl.BlockSpec((bn, d), lambda i: (i, 0))] + wspec,
        out_specs=[pl.BlockSpec((bn, _HP), lambda i: (i, 0)),
                   pl.BlockSpec((bn, _H), lambda i: (i, 0))],
        out_shape=[jax.ShapeDtypeStruct((n, _HP), jnp.float32),
                   jax.ShapeDtypeStruct((n, _H), jnp.float32)],
    )(features, w1, b1[None], w2, b2[None], ua, ub1[None])

    cnt = ep - e
    pad_src = jnp.arange(cnt, dtype=jnp.int32) % n
    pad_dst = n + jnp.arange(cnt, dtype=jnp.int32) % (np_ - n)
    src2d = jnp.concatenate([edges[1], pad_src]).reshape(-1, _B)
    dst2d = jnp.concatenate([edges[0], pad_dst]).reshape(-1, _B)
    w2d = jnp.concatenate(
        [edge_weights, jnp.zeros((cnt,), jnp.float32)]).reshape(-1, _B)
    iota = (jnp.arange(_NW, dtype=jnp.int32)[:, None] * _RT
            + jnp.arange(_RT, dtype=jnp.int32)[None, :])
    zeros = jnp.zeros((np_ // _NS, _HP), jnp.float32)
    acc = _make_sc_edge(np_)(msgs_p, src2d, dst2d, w2d, iota, zeros)

    wspec2 = [pl.BlockSpec(ws.shape, lambda i: (0,) * ws.ndim)
              for ws in (ub, uw2, ub2[None])]
    out = pl.pallas_call(
        _tc2_body,
        grid=(grid,),
        in_specs=[pl.BlockSpec((bn, _H), lambda i: (i, 0)),
                  pl.BlockSpec((_NC, bn, _HP), lambda i: (0, i, 0))] + wspec2,
        out_specs=pl.BlockSpec((bn, _H), lambda i: (i, 0)),
        out_shape=jax.ShapeDtypeStruct((n, _H), jnp.float32),
    )(part, acc, ub, uw2, ub2[None])
    return out


# speedup vs baseline: 2.1794x; 1.0006x over previous
import functools

import jax
import jax.numpy as jnp
from jax import lax
from jax.experimental import pallas as pl
from jax.experimental.pallas import tpu as pltpu
from jax.experimental.pallas import tpu_sc as plsc

_BN_EPS = 1e-3
_SQRT_HALF = 0.7071067811865476

_NC, _NS, _L = 2, 16, 16
_NW = _NC * _NS
_B = 256
_RT = 40
_CR = 1
_NB = 4
_H = 32
_HP = 40


def _gelu(x):
    return x * 0.5 * (1.0 + lax.erf(x * _SQRT_HALF))


def _fold_layer(p):
    scale = p["gamma"] / jnp.sqrt(p["var"] + _BN_EPS)
    shift = p["beta"] - p["mean"] * scale
    w = scale[:, None] * p["W"]
    b = shift @ p["W"] + p["b"]
    return w, b


def _tc1_body(f_ref, w1_ref, b1_ref, w2_ref, b2_ref, ua_ref, u1_ref,
              msgs_ref, part_ref):
    x = f_ref[...]
    n = x.shape[0]
    h1 = _gelu(jnp.dot(x, w1_ref[...], preferred_element_type=jnp.float32)
               + b1_ref[...])
    m = _gelu(jnp.dot(h1, w2_ref[...], preferred_element_type=jnp.float32)
              + b2_ref[...])
    col = lax.broadcasted_iota(jnp.int32, (n, _HP - _H), 1)
    tail = jnp.where(col == 0, 1.0, 0.0).astype(jnp.float32)
    msgs_ref[...] = jnp.concatenate([m, tail], axis=1)
    part_ref[...] = (jnp.dot(x, ua_ref[...], preferred_element_type=jnp.float32)
                     + u1_ref[...])


def _tc2_body(part_ref, acc_ref, ub_ref, u2_ref, b2_ref, out_ref):
    t = acc_ref[0] + acc_ref[1]
    s = t[:, :_H]
    c = t[:, _H:_H + 1]
    agg = s / jnp.maximum(c, 1.0)
    x1 = _gelu(part_ref[...]
               + jnp.dot(agg, ub_ref[...], preferred_element_type=jnp.float32))
    x2 = _gelu(jnp.dot(x1, u2_ref[...], preferred_element_type=jnp.float32)
               + b2_ref[...])
    ss = jnp.sum(x2 * x2, axis=-1, keepdims=True)
    out_ref[...] = x2 * lax.rsqrt(jnp.maximum(ss, 1e-12))


def _splat16(vec, j):
    return lax.gather(
        vec, jnp.full((_L, 1), j, jnp.int32),
        lax.GatherDimensionNumbers(offset_dims=(), collapsed_slice_dims=(0,),
                                   start_index_map=(0,)),
        (1,), mode=lax.GatherScatterMode.PROMISE_IN_BOUNDS)


@functools.lru_cache(maxsize=None)
def _make_sc_edge(np_):
    nch = _RT // _CR
    rpt = np_ // _NS

    @functools.partial(
        pl.kernel,
        out_type=jax.ShapeDtypeStruct((_NC, np_, _HP), jnp.float32),
        mesh=plsc.VectorSubcoreMesh(core_axis_name="c", subcore_axis_name="s"),
        compiler_params=pltpu.CompilerParams(use_tc_tiling_on_sc=False),
        scratch_types=[
            pltpu.VMEM((_RT,), jnp.int32),
            pltpu.VMEM((_RT, _B), jnp.int32),
            pltpu.VMEM((_RT, _B), jnp.int32),
            pltpu.VMEM((_RT, _B), jnp.float32),
        ] + [pltpu.VMEM((_CR * _B, _HP), jnp.float32)] * _NB
          + [pltpu.VMEM_SHARED((np_, _HP), jnp.float32)]
          + [pltpu.SemaphoreType.DMA] * (1 + 2 * _NB))
    def sc_edge(msgs_hbm, src_hbm, dst_hbm, w_hbm, iota_hbm, zero_hbm,
                acc_out,
                iidx_v, src_v, dst_v, w_v, *rest):
        bufs = rest[:_NB]
        acc_sh = rest[_NB]
        sem_e = rest[_NB + 1]
        gsems = rest[_NB + 2:_NB + 2 + _NB]
        ssems = rest[_NB + 2 + _NB:]
        cid = lax.axis_index("c")
        sid = lax.axis_index("s")
        wid = cid * _NS + sid

        r0 = sid * rpt
        pltpu.sync_copy(zero_hbm, acc_sh.at[pl.ds(r0, rpt)])
        pltpu.sync_copy(iota_hbm.at[wid], iidx_v)
        cps = [pltpu.async_copy(src_hbm.at[iidx_v], src_v, sem_e),
               pltpu.async_copy(dst_hbm.at[iidx_v], dst_v, sem_e),
               pltpu.async_copy(w_hbm.at[iidx_v], w_v, sem_e)]
        for cp in cps:
            cp.wait()
        plsc.subcore_barrier()

        def fire_gather(k, b):
            buf, sem = bufs[b], gsems[b]
            for j in range(_CR):
                pltpu.async_copy(msgs_hbm.at[src_v.at[k * _CR + j]],
                                 buf.at[pl.ds(j * _B, _B)], sem)

        def wait_gather(b):
            buf, sem = bufs[b], gsems[b]
            for j in range(_CR):
                pltpu.make_async_copy(msgs_hbm.at[src_v.at[0]],
                                      buf.at[pl.ds(j * _B, _B)], sem).wait()

        def fire_scatter(k, b):
            buf, sem = bufs[b], ssems[b]
            for j in range(_CR):
                pltpu.async_copy(buf.at[pl.ds(j * _B, _B)],
                                 acc_sh.at[dst_v.at[k * _CR + j]], sem,
                                 add=True)

        def wait_scatter(b):
            buf, sem = bufs[b], ssems[b]
            for j in range(_CR):
                pltpu.make_async_copy(buf.at[pl.ds(j * _B, _B)],
                                      acc_sh.at[dst_v.at[0]], sem).wait()

        def scale(k, b):
            buf = bufs[b]

            @plsc.parallel_loop(0, _CR * _B // _L, unroll=2)
            def grp_body(t):
                w16 = w_v[k * _CR + t // (_B // _L),
                          pl.ds((t % (_B // _L)) * _L, _L)]
                for l in range(_L):
                    ws = _splat16(w16, l)
                    r = t * _L + l
                    buf[r, pl.ds(0, _L)] = buf[r, pl.ds(0, _L)] * ws
                    buf[r, pl.ds(_L, _L)] = buf[r, pl.ds(_L, _L)] * ws

        nouter = nch // _NB
        fire_gather(0, 0)
        fire_gather(1, 1)
        fire_gather(2, 2)
        wait_gather(0); scale(0, 0); fire_scatter(0, 0)
        fire_gather(3, 3)
        wait_gather(1); scale(1, 1); fire_scatter(1, 1)
        wait_scatter(0); fire_gather(4, 0)
        wait_gather(2); scale(2, 2); fire_scatter(2, 2)
        wait_scatter(1); fire_gather(5, 1)
        wait_gather(3); scale(3, 3); fire_scatter(3, 3)

        def outer_body(o, carry):
            for i in range(_NB):
                k = o * _NB + i
                wait_scatter((i + 2) % _NB)
                fire_gather(k + 2, (i + 2) % _NB)
                wait_gather(i)
                scale(k, i)
                fire_scatter(k, i)
            return carry
        lax.fori_loop(1, nouter - 1, outer_body, 0)

        k0 = (nouter - 1) * _NB
        wait_scatter(2); fire_gather(k0 + 2, 2)
        wait_gather(0); scale(k0 + 0, 0); fire_scatter(k0 + 0, 0)
        wait_scatter(3); fire_gather(k0 + 3, 3)
        wait_gather(1); scale(k0 + 1, 1); fire_scatter(k0 + 1, 1)
        wait_scatter(0)
        wait_gather(2); scale(k0 + 2, 2); fire_scatter(k0 + 2, 2)
        wait_scatter(1)
        wait_gather(3); scale(k0 + 3, 3); fire_scatter(k0 + 3, 3)
        wait_scatter(2)
        wait_scatter(3)

        plsc.subcore_barrier()
        pltpu.sync_copy(acc_sh.at[pl.ds(r0, rpt)],
                        acc_out.at[cid, pl.ds(r0, rpt)])

    return sc_edge


def kernel(features, edges, edge_weights, params):
    n, d = features.shape
    e = edges.shape[1]
    np_ = ((n + 16 * 8 - 1) // (16 * 8)) * (16 * 8)
    ep = _NW * _RT * _B

    w1, b1 = _fold_layer(params["prepare"][0])
    w2, b2 = _fold_layer(params["prepare"][1])
    uw1, ub1 = _fold_layer(params["update"][0])
    uw2, ub2 = _fold_layer(params["update"][1])
    ua, ub = uw1[:d], uw1[d:]

    grid = 5
    bn = n // grid
    wspec = [pl.BlockSpec(ws.shape, lambda i: (0,) * ws.ndim)
             for ws in (w1, b1[None], w2, b2[None], ua, ub1[None])]
    msgs_p, part = pl.pallas_call(
        _tc1_body,
        grid=(grid,),
        in_specs=[pl.BlockSpec((bn, d), lambda i: (i, 0))] + wspec,
        out_specs=[pl.BlockSpec((bn, _HP), lambda i: (i, 0)),
                   pl.BlockSpec((bn, _H), lambda i: (i, 0))],
        out_shape=[jax.ShapeDtypeStruct((n, _HP), jnp.float32),
                   jax.ShapeDtypeStruct((n, _H), jnp.float32)],
    )(features, w1, b1[None], w2, b2[None], ua, ub1[None])

    cnt = ep - e
    pad_src = jnp.arange(cnt, dtype=jnp.int32) % n
    pad_dst = n + jnp.arange(cnt, dtype=jnp.int32) % (np_ - n)
    src2d = jnp.concatenate([edges[1], pad_src]).reshape(-1, _B)
    dst2d = jnp.concatenate([edges[0], pad_dst]).reshape(-1, _B)
    w2d = jnp.concatenate(
        [edge_weights, jnp.zeros((cnt,), jnp.float32)]).reshape(-1, _B)
    iota = (jnp.arange(_NW, dtype=jnp.int32)[:, None] * _RT
            + jnp.arange(_RT, dtype=jnp.int32)[None, :])
    zeros = jnp.zeros((np_ // _NS, _HP), jnp.float32)
    acc = _make_sc_edge(np_)(msgs_p, src2d, dst2d, w2d, iota, zeros)

    wspec2 = [pl.BlockSpec(ws.shape, lambda i: (0,) * ws.ndim)
              for ws in (ub, uw2, ub2[None])]
    out = pl.pallas_call(
        _tc2_body,
        grid=(grid,),
        in_specs=[pl.BlockSpec((bn, _H), lambda i: (i, 0)),
                  pl.BlockSpec((_NC, bn, _HP), lambda i: (0, i, 0))] + wspec2,
        out_specs=pl.BlockSpec((bn, _H), lambda i: (i, 0)),
        out_shape=jax.ShapeDtypeStruct((n, _H), jnp.float32),
    )(part, acc, ub, uw2, ub2[None])
    return out
